# baseline (speedup 1.0000x reference)
import jax
import jax.numpy as jnp
from jax import lax
from jax.experimental import pallas as pl
from jax.experimental.pallas import tpu as pltpu

_B, _L, _D, _H, _HID = 256, 8, 128, 2, 256
_BT = 32
_N = _BT * _L * _L
_R = _BT * _L


def _leaky(x):
    return jnp.maximum(x, 0.01 * x)


def _attn_kernel(el_ref, pk_ref,
                 w1c_ref, ws1b_ref, ws2_ref, ws2b_ref,
                 wc1b_ref, wc2_ref, wc2b_ref, out_ref):
    el = el_ref[...]
    el2 = el.reshape(_R, _D)

    pk = pk_ref[...].reshape(3, _N)
    pkt = jnp.transpose(pk, (1, 0))
    gie = pkt[:, 0:1]
    gio = pkt[:, 1:2]
    fr = lax.bitcast_convert_type(pkt[:, 2:3], jnp.float32)

    base = pl.program_id(0) * _R
    col = lax.broadcasted_iota(jnp.int32, (_N, _R), 1) + base
    ohe = (gie == col).astype(jnp.bfloat16)
    oho = (gio == col).astype(jnp.bfloat16)
    el2b = el2.astype(jnp.bfloat16)
    ae = jnp.dot(ohe, el2b, preferred_element_type=jnp.float32)
    ao = jnp.dot(oho, el2b, preferred_element_type=jnp.float32)
    aeb = ae.astype(jnp.bfloat16)
    aob = ao.astype(jnp.bfloat16)

    halp = (jnp.dot(aeb, w1c_ref[:_D, :], preferred_element_type=jnp.float32)
            + jnp.dot(aob, w1c_ref[_D:, :], preferred_element_type=jnp.float32))

    acc = jnp.zeros((_R, _D), jnp.float32)
    for h in range(_H):
        h1 = _leaky(halp[:, h * _HID:(h + 1) * _HID]
                    + ws1b_ref[h][None, :]).astype(jnp.bfloat16)
        aw = (jnp.dot(h1, ws2_ref[h].astype(jnp.bfloat16),
                      preferred_element_type=jnp.float32)
              + ws2b_ref[h][None, :])
        w = jnp.exp(aw) * fr

        c1 = _leaky(halp[:, (2 + h) * _HID:(3 + h) * _HID]
                    + wc1b_ref[h][None, :]).astype(jnp.bfloat16)
        c = (jnp.dot(c1, wc2_ref[h].astype(jnp.bfloat16),
                     preferred_element_type=jnp.float32)
             + wc2b_ref[h][None, :])

        cw = c * w
        wl = w * jnp.ones((1, _D), jnp.float32)
        num = jnp.sum(cw.reshape(_R, _L, _D), axis=1)
        den = jnp.sum(wl.reshape(_R, _L, _D), axis=1)
        acc = acc + num / den

    res = el2 + acc * (1.0 / _H)
    out_ref[...] = res.reshape(_BT, _L, _D)


def kernel(element, fraction, element_indices, Ws1_w, Ws1_b, Ws2_w, Ws2_b,
           Wc1_w, Wc1_b, Wc2_w, Wc2_b):
    b = element.shape[0]
    ntiles = b // _BT
    boff = (_L * jnp.arange(b, dtype=jnp.int32))[:, None]
    gidx = element_indices.astype(jnp.int32) + boff
    frbits = lax.bitcast_convert_type(jnp.tile(fraction, (1, _L)), jnp.int32)
    packed = jnp.stack([gidx[:, 0::2].reshape(ntiles, _N),
                        gidx[:, 1::2].reshape(ntiles, _N),
                        frbits.reshape(ntiles, _N)], axis=1)
    w1c = jnp.concatenate([Ws1_w[0], Ws1_w[1], Wc1_w[0], Wc1_w[1]],
                          axis=1).astype(jnp.bfloat16)

    full = lambda a: pl.BlockSpec(a.shape, lambda i: (0,) * a.ndim)
    out = pl.pallas_call(
        _attn_kernel,
        grid=(ntiles,),
        in_specs=[
            pl.BlockSpec((_BT, _L, _D), lambda i: (i, 0, 0)),
            pl.BlockSpec((1, 3, _N), lambda i: (i, 0, 0)),
            full(w1c), full(Ws1_b), full(Ws2_w), full(Ws2_b),
            full(Wc1_b), full(Wc2_w), full(Wc2_b),
        ],
        out_specs=pl.BlockSpec((_BT, _L, _D), lambda i: (i, 0, 0)),
        out_shape=jax.ShapeDtypeStruct((b, _L, _D), jnp.float32),
        compiler_params=pltpu.CompilerParams(
            dimension_semantics=("parallel",)),
    )(element, packed, w1c, Ws1_b, Ws2_w, Ws2_b, Wc1_b, Wc2_w, Wc2_b)
    return out


# speedup vs baseline: 1.0159x; 1.0159x over previous
import jax
import jax.numpy as jnp
from jax import lax
from jax.experimental import pallas as pl
from jax.experimental.pallas import tpu as pltpu

_B, _L, _D, _H, _HID = 256, 8, 128, 2, 256
_BT = 32
_N = _BT * _L * _L
_R = _BT * _L


def _leaky(x):
    return jnp.maximum(x, 0.01 * x)


def _attn_kernel(el_ref, pk_ref,
                 ws1_ref, ws1b_ref, ws2_ref, ws2b_ref,
                 wc1_ref, wc1b_ref, wc2_ref, wc2b_ref, out_ref):
    el = el_ref[...]
    el2 = el.reshape(_R, _D)

    pk = pk_ref[...].reshape(3, _N)
    pkt = jnp.transpose(pk, (1, 0))
    gie = pkt[:, 0:1]
    gio = pkt[:, 1:2]
    fr = lax.bitcast_convert_type(pkt[:, 2:3], jnp.float32)

    base = pl.program_id(0) * _R
    col = lax.broadcasted_iota(jnp.int32, (_N, _R), 1) + base
    ohe = (gie == col).astype(jnp.bfloat16)
    oho = (gio == col).astype(jnp.bfloat16)
    el2b = el2.astype(jnp.bfloat16)
    ae = jnp.dot(ohe, el2b, preferred_element_type=jnp.float32)
    ao = jnp.dot(oho, el2b, preferred_element_type=jnp.float32)
    aeb = ae.astype(jnp.bfloat16)
    aob = ao.astype(jnp.bfloat16)

    w1c = jnp.concatenate([ws1_ref[0], ws1_ref[1], wc1_ref[0], wc1_ref[1]],
                          axis=1).astype(jnp.bfloat16)
    halp = (jnp.dot(aeb, w1c[:_D, :], preferred_element_type=jnp.float32)
            + jnp.dot(aob, w1c[_D:, :], preferred_element_type=jnp.float32))

    acc = jnp.zeros((_R, _D), jnp.float32)
    for h in range(_H):
        h1 = _leaky(halp[:, h * _HID:(h + 1) * _HID]
                    + ws1b_ref[h][None, :]).astype(jnp.bfloat16)
        aw = (jnp.dot(h1, ws2_ref[h].astype(jnp.bfloat16),
                      preferred_element_type=jnp.float32)
              + ws2b_ref[h][None, :])
        w = jnp.exp(aw) * fr

        c1 = _leaky(halp[:, (2 + h) * _HID:(3 + h) * _HID]
                    + wc1b_ref[h][None, :]).astype(jnp.bfloat16)
        c = (jnp.dot(c1, wc2_ref[h].astype(jnp.bfloat16),
                     preferred_element_type=jnp.float32)
             + wc2b_ref[h][None, :])

        cw = c * w
        wl = w * jnp.ones((1, _D), jnp.float32)
        num = jnp.sum(cw.reshape(_R, _L, _D), axis=1)
        den = jnp.sum(wl.reshape(_R, _L, _D), axis=1)
        acc = acc + num / den

    res = el2 + acc * (1.0 / _H)
    out_ref[...] = res.reshape(_BT, _L, _D)


def kernel(element, fraction, element_indices, Ws1_w, Ws1_b, Ws2_w, Ws2_b,
           Wc1_w, Wc1_b, Wc2_w, Wc2_b):
    b = element.shape[0]
    ntiles = b // _BT
    boff = (_L * jnp.arange(b, dtype=jnp.int32))[:, None]
    gidx = element_indices.astype(jnp.int32) + boff
    frbits = lax.bitcast_convert_type(jnp.tile(fraction, (1, _L)), jnp.int32)
    packed = jnp.stack([gidx[:, 0::2].reshape(ntiles, _N),
                        gidx[:, 1::2].reshape(ntiles, _N),
                        frbits.reshape(ntiles, _N)], axis=1)
    full = lambda a: pl.BlockSpec(a.shape, lambda i: (0,) * a.ndim)
    out = pl.pallas_call(
        _attn_kernel,
        grid=(ntiles,),
        in_specs=[
            pl.BlockSpec((_BT, _L, _D), lambda i: (i, 0, 0)),
            pl.BlockSpec((1, 3, _N), lambda i: (i, 0, 0)),
            full(Ws1_w), full(Ws1_b), full(Ws2_w), full(Ws2_b),
            full(Wc1_w), full(Wc1_b), full(Wc2_w), full(Wc2_b),
        ],
        out_specs=pl.BlockSpec((_BT, _L, _D), lambda i: (i, 0, 0)),
        out_shape=jax.ShapeDtypeStruct((b, _L, _D), jnp.float32),
        compiler_params=pltpu.CompilerParams(
            dimension_semantics=("parallel",)),
    )(element, packed, Ws1_w, Ws1_b, Ws2_w, Ws2_b, Wc1_w, Wc1_b, Wc2_w, Wc2_b)
    return out
